# baseline (speedup 1.0000x reference)
import functools

import jax
import jax.numpy as jnp
import numpy as np
from jax.experimental import pallas as pl
from jax.experimental.pallas import tpu as pltpu

_B = 1024
_P = 64

_SPACE = np.array([8000.0, 8000.0, 2000.0], np.float32)
_VOX = np.array([80.0, 80.0, 20.0], np.float32)
_CENTER = np.array([0.0, 0.0, 1000.0], np.float32)
_SCALE = _SPACE / (_VOX - 1.0)
_BIAS = _CENTER - _SPACE / 2.0
_MIN_SCORE = 0.3

_PB = 64
_GRID = _P // _PB


def _body(idx_ref, conf_ref, bbox_ref, out_ref):
    sx, sy, sz = float(_SCALE[0]), float(_SCALE[1]), float(_SCALE[2])
    bx, by, bz = float(_BIAS[0]), float(_BIAS[1]), float(_BIAS[2])
    idxf = idx_ref[...].astype(jnp.float32)
    out_ref[0] = idxf[0] * sx + bx
    out_ref[1] = idxf[1] * sy + by
    out_ref[2] = idxf[2] * sz + bz
    cf = conf_ref[...]
    out_ref[3] = (cf > _MIN_SCORE).astype(jnp.float32) - 1.0
    out_ref[4] = cf
    out_ref[5] = bbox_ref[:, 0, :]
    out_ref[6] = bbox_ref[:, 1, :]


@jax.jit
def _proposal_tc(idx_t, conf_t, bbox_t):
    return pl.pallas_call(
        _body,
        grid=(_GRID,),
        in_specs=[
            pl.BlockSpec((3, _PB, _B), lambda i: (0, i, 0)),
            pl.BlockSpec((_PB, _B), lambda i: (i, 0)),
            pl.BlockSpec((_PB, 2, _B), lambda i: (i, 0, 0)),
        ],
        out_specs=pl.BlockSpec((7, _PB, _B), lambda i: (0, i, 0)),
        out_shape=jax.ShapeDtypeStruct((7, _P, _B), jnp.float32),
    )(idx_t, conf_t, bbox_t)


def kernel(topk_index, topk_confs, match_bbox_preds, meta):
    del meta
    idx_t = jnp.transpose(topk_index, (2, 1, 0))
    conf_t = jnp.transpose(topk_confs, (1, 0))
    bbox_t = jnp.transpose(match_bbox_preds, (1, 2, 0))
    out_t = _proposal_tc(idx_t, conf_t, bbox_t)
    return jnp.transpose(out_t, (2, 1, 0))
